# baseline (speedup 1.0000x reference)
import functools

import jax
import jax.numpy as jnp
from jax import lax
from jax.experimental import pallas as pl
from jax.experimental.pallas import tpu as pltpu
from jax.experimental.pallas import tpu_sc as plsc

NC = 2
NS = 16
LANES = 16


def _radial_body(r_ref, w_ref, o_ref):
    x = jnp.dot(r_ref[...], w_ref[...], preferred_element_type=jnp.float32)
    h = x.shape[1] // 2
    lo = lax.bitcast_convert_type(x[:, :h].astype(jnp.bfloat16),
                                  jnp.uint16).astype(jnp.uint32)
    hi = lax.bitcast_convert_type(x[:, h:].astype(jnp.bfloat16),
                                  jnp.uint16).astype(jnp.uint32)
    o_ref[...] = lax.bitcast_convert_type(lo | (hi << 16), jnp.int32)


def _proj_radial(R, W):
    E, DE = R.shape
    D = W.shape[1]
    BE = 4000
    assert E % BE == 0 and D % 2 == 0
    return pl.pallas_call(
        _radial_body,
        grid=(E // BE,),
        in_specs=[
            pl.BlockSpec((BE, DE), lambda i: (i, 0)),
            pl.BlockSpec((DE, D), lambda i: (0, 0)),
        ],
        out_specs=pl.BlockSpec((BE, D // 2), lambda i: (i, 0)),
        out_shape=jax.ShapeDtypeStruct((E, D // 2), jnp.int32),
    )(R, W)


def _sc_edge_aggregate(node_attr, edge_center, edge_neigh, proj_radial):
    N, D = node_attr.shape
    E = edge_center.shape[0]
    NW = NC * NS
    EPW = E // NW
    G = 96
    NG = EPW // G
    TAIL = EPW - NG * G
    assert NG >= 6 and NG % 2 == 0 and 0 < TAIL <= G and TAIL % 8 == 0
    RPT = (N // NS) // 8 * 8
    REM = N - RPT * NS
    ZF = RPT // G
    ZREM = RPT - ZF * G
    assert REM % 8 == 0 and ZREM % 8 == 0 and (RPT * NS) % 8 == 0

    mesh = plsc.VectorSubcoreMesh(core_axis_name="c", subcore_axis_name="s",
                                  num_cores=NC, num_subcores=NS)

    @functools.partial(
        pl.kernel,
        out_type=jax.ShapeDtypeStruct((NC, N, D), jnp.float32),
        mesh=mesh,
        scratch_types=[
            pltpu.VMEM((G,), jnp.int32),
            pltpu.VMEM((G,), jnp.int32),
            pltpu.VMEM((G,), jnp.int32),
            pltpu.VMEM((G,), jnp.int32),
            pltpu.VMEM((TAIL,), jnp.int32),
            pltpu.VMEM((TAIL,), jnp.int32),
            pltpu.VMEM((G,), jnp.int32),
            pltpu.VMEM((G,), jnp.int32),
            pltpu.VMEM((G, D), jnp.float32),
            pltpu.VMEM((G, D), jnp.float32),
            pltpu.VMEM((G, D // 2), jnp.int32),
            pltpu.VMEM((G, D // 2), jnp.int32),
            pltpu.VMEM_SHARED((N, D), jnp.float32),
            pltpu.SemaphoreType.DMA,
            pltpu.SemaphoreType.DMA,
            pltpu.SemaphoreType.DMA,
            pltpu.SemaphoreType.DMA,
            pltpu.SemaphoreType.DMA,
            pltpu.SemaphoreType.DMA,
            pltpu.SemaphoreType.DMA,
            pltpu.SemaphoreType.DMA,
        ],
    )
    def sck(node_hbm, ec_hbm, en_hbm, rad_hbm, out_hbm,
            nidx0, nidx1, cidx0, cidx1, tnidx, tcidx, scidx0, scidx1,
            rows0, rows1, rad0, rad1, macc,
            si0, si1, sg0, sg1, sr0, sr1, ss0, ss1):
        sem_i, sem_g, sem_r, sem_s = [si0, si1], [sg0, sg1], [sr0, sr1], [ss0, ss1]
        nidx_s, cidx_s = [nidx0, nidx1], [cidx0, cidx1]
        scidx_s = [scidx0, scidx1]
        rows_s, rad_s = [rows0, rows1], [rad0, rad1]
        cid = lax.axis_index("c")
        sid = lax.axis_index("s")
        wid = sid * NC + cid
        base0 = wid * EPW

        def issue_idx(g, b):
            off = base0 + g * G
            pltpu.async_copy(ec_hbm.at[pl.ds(off, G)], cidx_s[b], sem_i[b])
            pltpu.async_copy(en_hbm.at[pl.ds(off, G)], nidx_s[b], sem_i[b])

        def wait_idx(b):
            pltpu.make_async_copy(ec_hbm.at[pl.ds(0, G)], cidx_s[b],
                                  sem_i[b]).wait()
            pltpu.make_async_copy(en_hbm.at[pl.ds(0, G)], nidx_s[b],
                                  sem_i[b]).wait()

        def issue_fetch(g, b):
            pltpu.async_copy(node_hbm.at[nidx_s[b]], rows_s[b], sem_g[b])
            pltpu.async_copy(rad_hbm.at[pl.ds(base0 + g * G, G)], rad_s[b],
                             sem_r[b])

        def wait_fetch(b):
            pltpu.make_async_copy(node_hbm.at[nidx_s[b]], rows_s[b],
                                  sem_g[b]).wait()
            pltpu.make_async_copy(rad_hbm.at[pl.ds(0, G)], rad_s[b],
                                  sem_r[b]).wait()

        HMASK = jnp.full((LANES,), -65536, jnp.int32)
        H = D // 2

        def mul_row(rows_b, rad_b, r):
            for c in range(H // LANES):
                v = rad_b[r, pl.ds(c * LANES, LANES)]
                lo = lax.bitcast_convert_type(v << 16, jnp.float32)
                hi = lax.bitcast_convert_type(v & HMASK, jnp.float32)
                sle = pl.ds(c * LANES, LANES)
                slo = pl.ds(H + c * LANES, LANES)
                rows_b[r, sle] = rows_b[r, sle] * lo
                rows_b[r, slo] = rows_b[r, slo] * hi

        def mul(b):
            rows_b, rad_b = rows_s[b], rad_s[b]

            @plsc.parallel_loop(0, G, 1, unroll=8)
            def _(r):
                mul_row(rows_b, rad_b, r)

        def snap(b):
            src, dst = cidx_s[b], scidx_s[b]

            @plsc.parallel_loop(0, G // LANES, 1, unroll=2)
            def _(k):
                sl = pl.ds(k * LANES, LANES)
                dst[sl] = src[sl]

        def issue_scat(b):
            pltpu.async_copy(rows_s[b], macc.at[scidx_s[b]], sem_s[b],
                             add=True)

        def wait_scat(b):
            pltpu.make_async_copy(rows_s[b], macc.at[scidx_s[b]],
                                  sem_s[b]).wait()

        zero = jnp.zeros((LANES,), jnp.float32)

        def zfill(r, carry):
            for c in range(D // LANES):
                rows0[r, pl.ds(c * LANES, LANES)] = zero
            return carry
        lax.fori_loop(0, G, zfill, 0)

        def zcopy(k, carry):
            pltpu.sync_copy(rows0, macc.at[pl.ds(sid * RPT + k * G, G)])
            return carry
        lax.fori_loop(0, ZF, zcopy, 0)
        pltpu.sync_copy(rows0.at[pl.ds(0, ZREM)],
                        macc.at[pl.ds(sid * RPT + ZF * G, ZREM)])

        @pl.when(sid == 0)
        def _():
            pltpu.sync_copy(rows0.at[pl.ds(0, REM)],
                            macc.at[pl.ds(RPT * NS, REM)])

        plsc.subcore_barrier()

        issue_idx(0, 0)
        issue_idx(1, 1)
        wait_idx(0)
        issue_fetch(0, 0)
        wait_fetch(0)
        snap(0)
        issue_idx(2, 0)
        wait_idx(1)
        issue_fetch(1, 1)
        mul(0)
        issue_scat(0)
        wait_fetch(1)
        snap(1)
        issue_idx(3, 1)
        wait_idx(0)
        wait_scat(0)
        issue_fetch(2, 0)
        mul(1)
        issue_scat(1)

        def steady(g, b):
            ob = 1 - b
            wait_fetch(b)
            snap(b)
            issue_idx(g + 2, b)
            wait_idx(ob)
            wait_scat(ob)
            issue_fetch(g + 1, ob)
            mul(b)
            issue_scat(b)

        def pair(p, carry):
            steady(2 * p, 0)
            steady(2 * p + 1, 1)
            return carry
        lax.fori_loop(1, (NG - 2) // 2, pair, 0)

        wait_fetch(0)
        snap(0)
        wait_idx(1)
        wait_scat(1)
        issue_fetch(NG - 1, 1)
        mul(0)
        issue_scat(0)

        wait_fetch(1)
        snap(1)
        mul(1)
        wait_scat(0)
        issue_scat(1)

        toff = base0 + NG * G
        pltpu.sync_copy(ec_hbm.at[pl.ds(toff, TAIL)], tcidx)
        pltpu.sync_copy(en_hbm.at[pl.ds(toff, TAIL)], tnidx)
        pltpu.async_copy(node_hbm.at[tnidx], rows0.at[pl.ds(0, TAIL)],
                         sg0).wait()
        pltpu.sync_copy(rad_hbm.at[pl.ds(toff, TAIL)],
                        rad0.at[pl.ds(0, TAIL)])

        def tmul(r, carry):
            mul_row(rows0, rad0, r)
            return carry
        lax.fori_loop(0, TAIL, tmul, 0)
        pltpu.sync_copy(rows0.at[pl.ds(0, TAIL)], macc.at[tcidx], add=True)

        wait_scat(1)
        plsc.subcore_barrier()

        r0 = sid * RPT
        pltpu.sync_copy(macc.at[pl.ds(r0, RPT)],
                        out_hbm.at[cid, pl.ds(r0, RPT)])

        @pl.when(sid == 0)
        def _():
            pltpu.sync_copy(macc.at[pl.ds(RPT * NS, REM)],
                            out_hbm.at[cid, pl.ds(RPT * NS, REM)])

    return sck(node_attr, edge_center, edge_neigh, proj_radial)


def _combine_body(na_ref, p_ref, wc_ref, wt_ref, wb_ref, g_ref, b_ref, o_ref):
    a = na_ref[...] * wc_ref[...]
    m = p_ref[0] + p_ref[1]
    x = (jnp.dot(a, wt_ref[...], preferred_element_type=jnp.float32)
         + jnp.dot(m, wb_ref[...], preferred_element_type=jnp.float32))
    mu = jnp.mean(x, axis=-1, keepdims=True)
    xc = x - mu
    var = jnp.mean(xc * xc, axis=-1, keepdims=True)
    o_ref[...] = xc * lax.rsqrt(var + 1e-5) * g_ref[...] + b_ref[...]


def _combine(node_attr, partials, wc_diag, W_top, W_bot, gamma, beta):
    N, D = node_attr.shape
    BN = 400
    assert N % BN == 0
    return pl.pallas_call(
        _combine_body,
        grid=(N // BN,),
        in_specs=[
            pl.BlockSpec((BN, D), lambda i: (i, 0)),
            pl.BlockSpec((NC, BN, D), lambda i: (0, i, 0)),
            pl.BlockSpec((1, D), lambda i: (0, 0)),
            pl.BlockSpec((D, D), lambda i: (0, 0)),
            pl.BlockSpec((D, D), lambda i: (0, 0)),
            pl.BlockSpec((1, D), lambda i: (0, 0)),
            pl.BlockSpec((1, D), lambda i: (0, 0)),
        ],
        out_specs=pl.BlockSpec((BN, D), lambda i: (i, 0)),
        out_shape=jax.ShapeDtypeStruct((N, D), jnp.float32),
    )(node_attr, partials, wc_diag.reshape(1, D), W_top, W_bot,
      gamma.reshape(1, D), beta.reshape(1, D))


def kernel(node_attr, edge_index, edge_radial_attrs, W_node, W_center_node,
           W_concat_node, W_edge, ln_gamma, ln_beta):
    D = node_attr.shape[1]
    wnode = jnp.diagonal(W_node)
    wcenter = jnp.diagonal(W_center_node)
    perm = jnp.concatenate([jnp.arange(0, D, 2), jnp.arange(1, D, 2)])
    node_perm = node_attr[:, perm]
    Wp = (W_edge * wnode[None, :])[:, perm]
    rad_packed = _proj_radial(edge_radial_attrs, Wp)
    ei = edge_index.astype(jnp.int32)
    partials = _sc_edge_aggregate(node_perm, ei[0], ei[1], rad_packed)
    W_top = W_concat_node[:D]
    W_bot = W_concat_node[D:][perm, :]
    return _combine(node_attr, partials, wcenter, W_top, W_bot,
                    ln_gamma, ln_beta)


# speedup vs baseline: 1.0698x; 1.0698x over previous
import functools

import jax
import jax.numpy as jnp
from jax import lax
from jax.experimental import pallas as pl
from jax.experimental.pallas import tpu as pltpu
from jax.experimental.pallas import tpu_sc as plsc

NC = 2
NS = 16
LANES = 16


def _radial_body(r_ref, w_ref, o_ref):
    x = jnp.dot(r_ref[...], w_ref[...], preferred_element_type=jnp.float32)
    h = x.shape[1] // 2
    lo = lax.bitcast_convert_type(x[:, :h].astype(jnp.bfloat16),
                                  jnp.uint16).astype(jnp.uint32)
    hi = lax.bitcast_convert_type(x[:, h:].astype(jnp.bfloat16),
                                  jnp.uint16).astype(jnp.uint32)
    o_ref[...] = lax.bitcast_convert_type(lo | (hi << 16), jnp.int32)


def _proj_radial(R, W):
    E, DE = R.shape
    D = W.shape[1]
    BE = 4000
    assert E % BE == 0 and D % 2 == 0
    return pl.pallas_call(
        _radial_body,
        grid=(E // BE,),
        in_specs=[
            pl.BlockSpec((BE, DE), lambda i: (i, 0)),
            pl.BlockSpec((DE, D), lambda i: (0, 0)),
        ],
        out_specs=pl.BlockSpec((BE, D // 2), lambda i: (i, 0)),
        out_shape=jax.ShapeDtypeStruct((E, D // 2), jnp.int32),
    )(R, W)


def _sc_edge_aggregate(node_attr, edge_center, edge_neigh, proj_radial):
    N, D = node_attr.shape
    E = edge_center.shape[0]
    NW = NC * NS
    EPW = E // NW
    G = 64
    NG = EPW // G
    TAIL = EPW - NG * G
    assert (NG >= 6 and (NG - 6) % 3 == 0 and 0 < TAIL <= G
            and TAIL % 8 == 0 and G % LANES == 0)
    RPT = (N // NS) // 8 * 8
    REM = N - RPT * NS
    ZF = RPT // G
    ZREM = RPT - ZF * G
    assert REM % 8 == 0 and ZREM % 8 == 0 and (RPT * NS) % 8 == 0

    mesh = plsc.VectorSubcoreMesh(core_axis_name="c", subcore_axis_name="s",
                                  num_cores=NC, num_subcores=NS)

    NSLOT = 3
    scratch = []
    for _ in range(NSLOT):
        scratch += [
            pltpu.VMEM((G,), jnp.int32),
            pltpu.VMEM((G,), jnp.int32),
            pltpu.VMEM((G,), jnp.int32),
            pltpu.VMEM((G, D), jnp.float32),
            pltpu.VMEM((G, D // 2), jnp.int32),
        ]
    scratch += [
        pltpu.VMEM((TAIL,), jnp.int32),
        pltpu.VMEM((TAIL,), jnp.int32),
        pltpu.VMEM_SHARED((N, D), jnp.float32),
    ]
    scratch += [pltpu.SemaphoreType.DMA] * (4 * NSLOT)

    @functools.partial(
        pl.kernel,
        out_type=jax.ShapeDtypeStruct((NC, N, D), jnp.float32),
        mesh=mesh,
        scratch_types=scratch,
    )
    def sck(node_hbm, ec_hbm, en_hbm, rad_hbm, out_hbm, *sc):
        slot_refs, rest = sc[:5 * NSLOT], sc[5 * NSLOT:]
        nidx_s = [slot_refs[5 * i + 0] for i in range(NSLOT)]
        cidx_s = [slot_refs[5 * i + 1] for i in range(NSLOT)]
        scidx_s = [slot_refs[5 * i + 2] for i in range(NSLOT)]
        rows_s = [slot_refs[5 * i + 3] for i in range(NSLOT)]
        rad_s = [slot_refs[5 * i + 4] for i in range(NSLOT)]
        tnidx, tcidx, macc = rest[0], rest[1], rest[2]
        sems = rest[3:]
        sem_i = list(sems[0:NSLOT])
        sem_g = list(sems[NSLOT:2 * NSLOT])
        sem_r = list(sems[2 * NSLOT:3 * NSLOT])
        sem_s = list(sems[3 * NSLOT:4 * NSLOT])
        rows0, rad0 = rows_s[0], rad_s[0]
        cid = lax.axis_index("c")
        sid = lax.axis_index("s")
        wid = sid * NC + cid
        base0 = wid * EPW

        def issue_idx(g, b):
            off = base0 + g * G
            pltpu.async_copy(ec_hbm.at[pl.ds(off, G)], cidx_s[b], sem_i[b])
            pltpu.async_copy(en_hbm.at[pl.ds(off, G)], nidx_s[b], sem_i[b])

        def wait_idx(b):
            pltpu.make_async_copy(ec_hbm.at[pl.ds(0, G)], cidx_s[b],
                                  sem_i[b]).wait()
            pltpu.make_async_copy(en_hbm.at[pl.ds(0, G)], nidx_s[b],
                                  sem_i[b]).wait()

        def issue_fetch(g, b):
            pltpu.async_copy(node_hbm.at[nidx_s[b]], rows_s[b], sem_g[b])
            pltpu.async_copy(rad_hbm.at[pl.ds(base0 + g * G, G)], rad_s[b],
                             sem_r[b])

        def wait_fetch(b):
            pltpu.make_async_copy(node_hbm.at[nidx_s[b]], rows_s[b],
                                  sem_g[b]).wait()
            pltpu.make_async_copy(rad_hbm.at[pl.ds(0, G)], rad_s[b],
                                  sem_r[b]).wait()

        HMASK = jnp.full((LANES,), -65536, jnp.int32)
        H = D // 2

        def mul_row(rows_b, rad_b, r):
            for c in range(H // LANES):
                v = rad_b[r, pl.ds(c * LANES, LANES)]
                lo = lax.bitcast_convert_type(v << 16, jnp.float32)
                hi = lax.bitcast_convert_type(v & HMASK, jnp.float32)
                sle = pl.ds(c * LANES, LANES)
                slo = pl.ds(H + c * LANES, LANES)
                rows_b[r, sle] = rows_b[r, sle] * lo
                rows_b[r, slo] = rows_b[r, slo] * hi

        def mul(b):
            rows_b, rad_b = rows_s[b], rad_s[b]

            @plsc.parallel_loop(0, G, 1, unroll=8)
            def _(r):
                mul_row(rows_b, rad_b, r)

        def snap(b):
            src, dst = cidx_s[b], scidx_s[b]

            @plsc.parallel_loop(0, G // LANES, 1, unroll=2)
            def _(k):
                sl = pl.ds(k * LANES, LANES)
                dst[sl] = src[sl]

        def issue_scat(b):
            pltpu.async_copy(rows_s[b], macc.at[scidx_s[b]], sem_s[b],
                             add=True)

        def wait_scat(b):
            pltpu.make_async_copy(rows_s[b], macc.at[scidx_s[b]],
                                  sem_s[b]).wait()

        zero = jnp.zeros((LANES,), jnp.float32)

        def zfill(r, carry):
            for c in range(D // LANES):
                rows0[r, pl.ds(c * LANES, LANES)] = zero
            return carry
        lax.fori_loop(0, G, zfill, 0)

        def zcopy(k, carry):
            pltpu.sync_copy(rows0, macc.at[pl.ds(sid * RPT + k * G, G)])
            return carry
        lax.fori_loop(0, ZF, zcopy, 0)
        pltpu.sync_copy(rows0.at[pl.ds(0, ZREM)],
                        macc.at[pl.ds(sid * RPT + ZF * G, ZREM)])

        @pl.when(sid == 0)
        def _():
            pltpu.sync_copy(rows0.at[pl.ds(0, REM)],
                            macc.at[pl.ds(RPT * NS, REM)])

        plsc.subcore_barrier()

        def ops(g, b, has_idx=True, has_fetch=True, has_drain=True):
            bf = (b + 2) % NSLOT
            wait_fetch(b)
            snap(b)
            if has_idx:
                issue_idx(g + NSLOT, b)
            if has_fetch:
                wait_idx(bf)
            if has_drain:
                wait_scat(bf)
            if has_fetch:
                issue_fetch(g + 2, bf)
            mul(b)
            issue_scat(b)

        issue_idx(0, 0)
        issue_idx(1, 1)
        issue_idx(2, 2)
        wait_idx(0)
        issue_fetch(0, 0)
        wait_idx(1)
        issue_fetch(1, 1)
        ops(0, 0, has_drain=False)
        ops(1, 1)
        ops(2, 2)

        def trio(p, carry):
            ops(3 * p, 0)
            ops(3 * p + 1, 1)
            ops(3 * p + 2, 2)
            return carry
        lax.fori_loop(1, (NG - 3) // 3, trio, 0)

        ops(NG - 3, 0, has_idx=False)
        ops(NG - 2, 1, has_idx=False, has_fetch=False)
        ops(NG - 1, 2, has_idx=False, has_fetch=False)

        toff = base0 + NG * G
        pltpu.sync_copy(ec_hbm.at[pl.ds(toff, TAIL)], tcidx)
        pltpu.sync_copy(en_hbm.at[pl.ds(toff, TAIL)], tnidx)
        pltpu.async_copy(node_hbm.at[tnidx], rows0.at[pl.ds(0, TAIL)],
                         sem_g[0]).wait()
        pltpu.sync_copy(rad_hbm.at[pl.ds(toff, TAIL)],
                        rad0.at[pl.ds(0, TAIL)])

        def tmul(r, carry):
            mul_row(rows0, rad0, r)
            return carry
        lax.fori_loop(0, TAIL, tmul, 0)
        pltpu.sync_copy(rows0.at[pl.ds(0, TAIL)], macc.at[tcidx], add=True)

        wait_scat(2)
        plsc.subcore_barrier()

        r0 = sid * RPT
        pltpu.sync_copy(macc.at[pl.ds(r0, RPT)],
                        out_hbm.at[cid, pl.ds(r0, RPT)])

        @pl.when(sid == 0)
        def _():
            pltpu.sync_copy(macc.at[pl.ds(RPT * NS, REM)],
                            out_hbm.at[cid, pl.ds(RPT * NS, REM)])

    return sck(node_attr, edge_center, edge_neigh, proj_radial)


def _combine_body(na_ref, p_ref, wc_ref, wt_ref, wb_ref, g_ref, b_ref, o_ref):
    a = na_ref[...] * wc_ref[...]
    m = p_ref[0] + p_ref[1]
    x = (jnp.dot(a, wt_ref[...], preferred_element_type=jnp.float32)
         + jnp.dot(m, wb_ref[...], preferred_element_type=jnp.float32))
    mu = jnp.mean(x, axis=-1, keepdims=True)
    xc = x - mu
    var = jnp.mean(xc * xc, axis=-1, keepdims=True)
    o_ref[...] = xc * lax.rsqrt(var + 1e-5) * g_ref[...] + b_ref[...]


def _combine(node_attr, partials, wc_diag, W_top, W_bot, gamma, beta):
    N, D = node_attr.shape
    BN = 400
    assert N % BN == 0
    return pl.pallas_call(
        _combine_body,
        grid=(N // BN,),
        in_specs=[
            pl.BlockSpec((BN, D), lambda i: (i, 0)),
            pl.BlockSpec((NC, BN, D), lambda i: (0, i, 0)),
            pl.BlockSpec((1, D), lambda i: (0, 0)),
            pl.BlockSpec((D, D), lambda i: (0, 0)),
            pl.BlockSpec((D, D), lambda i: (0, 0)),
            pl.BlockSpec((1, D), lambda i: (0, 0)),
            pl.BlockSpec((1, D), lambda i: (0, 0)),
        ],
        out_specs=pl.BlockSpec((BN, D), lambda i: (i, 0)),
        out_shape=jax.ShapeDtypeStruct((N, D), jnp.float32),
    )(node_attr, partials, wc_diag.reshape(1, D), W_top, W_bot,
      gamma.reshape(1, D), beta.reshape(1, D))


def kernel(node_attr, edge_index, edge_radial_attrs, W_node, W_center_node,
           W_concat_node, W_edge, ln_gamma, ln_beta):
    D = node_attr.shape[1]
    wnode = jnp.diagonal(W_node)
    wcenter = jnp.diagonal(W_center_node)
    perm = jnp.concatenate([jnp.arange(0, D, 2), jnp.arange(1, D, 2)])
    node_perm = node_attr[:, perm]
    Wp = (W_edge * wnode[None, :])[:, perm]
    rad_packed = _proj_radial(edge_radial_attrs, Wp)
    ei = edge_index.astype(jnp.int32)
    partials = _sc_edge_aggregate(node_perm, ei[0], ei[1], rad_packed)
    W_top = W_concat_node[:D]
    W_bot = W_concat_node[D:][perm, :]
    return _combine(node_attr, partials, wcenter, W_top, W_bot,
                    ln_gamma, ln_beta)


# speedup vs baseline: 1.0717x; 1.0018x over previous
import functools

import jax
import jax.numpy as jnp
from jax import lax
from jax.experimental import pallas as pl
from jax.experimental.pallas import tpu as pltpu
from jax.experimental.pallas import tpu_sc as plsc

NC = 2
NS = 16
LANES = 16


def _radial_body(r_ref, w_ref, o_ref):
    x = jnp.dot(r_ref[...], w_ref[...], preferred_element_type=jnp.float32)
    h = x.shape[1] // 2
    u = lax.bitcast_convert_type(x, jnp.uint32) + jnp.uint32(0x8000)
    lo = u[:, :h] >> 16
    hi = u[:, h:] & jnp.uint32(0xFFFF0000)
    o_ref[...] = lax.bitcast_convert_type(lo | hi, jnp.int32)


def _proj_radial(R, W):
    E, DE = R.shape
    D = W.shape[1]
    BE = 4000
    assert E % BE == 0 and D % 2 == 0
    return pl.pallas_call(
        _radial_body,
        grid=(E // BE,),
        in_specs=[
            pl.BlockSpec((BE, DE), lambda i: (i, 0)),
            pl.BlockSpec((DE, D), lambda i: (0, 0)),
        ],
        out_specs=pl.BlockSpec((BE, D // 2), lambda i: (i, 0)),
        out_shape=jax.ShapeDtypeStruct((E, D // 2), jnp.int32),
    )(R, W)


def _sc_edge_aggregate(node_attr, edge_center, edge_neigh, proj_radial):
    N, D = node_attr.shape
    E = edge_center.shape[0]
    NW = NC * NS
    EPW = E // NW
    G = 64
    NG = EPW // G
    TAIL = EPW - NG * G
    assert (NG >= 6 and (NG - 6) % 3 == 0 and 0 < TAIL <= G
            and TAIL % 8 == 0 and G % LANES == 0)
    RPT = (N // NS) // 8 * 8
    REM = N - RPT * NS
    ZF = RPT // G
    ZREM = RPT - ZF * G
    assert REM % 8 == 0 and ZREM % 8 == 0 and (RPT * NS) % 8 == 0

    mesh = plsc.VectorSubcoreMesh(core_axis_name="c", subcore_axis_name="s",
                                  num_cores=NC, num_subcores=NS)

    NSLOT = 3
    scratch = []
    for _ in range(NSLOT):
        scratch += [
            pltpu.VMEM((G,), jnp.int32),
            pltpu.VMEM((G,), jnp.int32),
            pltpu.VMEM((G,), jnp.int32),
            pltpu.VMEM((G, D), jnp.float32),
            pltpu.VMEM((G, D // 2), jnp.int32),
        ]
    scratch += [
        pltpu.VMEM((TAIL,), jnp.int32),
        pltpu.VMEM((TAIL,), jnp.int32),
        pltpu.VMEM_SHARED((N, D), jnp.float32),
    ]
    scratch += [pltpu.SemaphoreType.DMA] * (4 * NSLOT)

    @functools.partial(
        pl.kernel,
        out_type=jax.ShapeDtypeStruct((NC, N, D), jnp.float32),
        mesh=mesh,
        scratch_types=scratch,
    )
    def sck(node_hbm, ec_hbm, en_hbm, rad_hbm, out_hbm, *sc):
        slot_refs, rest = sc[:5 * NSLOT], sc[5 * NSLOT:]
        nidx_s = [slot_refs[5 * i + 0] for i in range(NSLOT)]
        cidx_s = [slot_refs[5 * i + 1] for i in range(NSLOT)]
        scidx_s = [slot_refs[5 * i + 2] for i in range(NSLOT)]
        rows_s = [slot_refs[5 * i + 3] for i in range(NSLOT)]
        rad_s = [slot_refs[5 * i + 4] for i in range(NSLOT)]
        tnidx, tcidx, macc = rest[0], rest[1], rest[2]
        sems = rest[3:]
        sem_i = list(sems[0:NSLOT])
        sem_g = list(sems[NSLOT:2 * NSLOT])
        sem_r = list(sems[2 * NSLOT:3 * NSLOT])
        sem_s = list(sems[3 * NSLOT:4 * NSLOT])
        rows0, rad0 = rows_s[0], rad_s[0]
        cid = lax.axis_index("c")
        sid = lax.axis_index("s")
        wid = sid * NC + cid
        base0 = wid * EPW

        def issue_idx(g, b):
            off = base0 + g * G
            pltpu.async_copy(ec_hbm.at[pl.ds(off, G)], cidx_s[b], sem_i[b])
            pltpu.async_copy(en_hbm.at[pl.ds(off, G)], nidx_s[b], sem_i[b])

        def wait_idx(b):
            pltpu.make_async_copy(ec_hbm.at[pl.ds(0, G)], cidx_s[b],
                                  sem_i[b]).wait()
            pltpu.make_async_copy(en_hbm.at[pl.ds(0, G)], nidx_s[b],
                                  sem_i[b]).wait()

        def issue_fetch(g, b):
            pltpu.async_copy(node_hbm.at[nidx_s[b]], rows_s[b], sem_g[b])
            pltpu.async_copy(rad_hbm.at[pl.ds(base0 + g * G, G)], rad_s[b],
                             sem_r[b])

        def wait_fetch(b):
            pltpu.make_async_copy(node_hbm.at[nidx_s[b]], rows_s[b],
                                  sem_g[b]).wait()
            pltpu.make_async_copy(rad_hbm.at[pl.ds(0, G)], rad_s[b],
                                  sem_r[b]).wait()

        HMASK = jnp.full((LANES,), -65536, jnp.int32)
        H = D // 2

        def mul_row(rows_b, rad_b, r):
            for c in range(H // LANES):
                v = rad_b[r, pl.ds(c * LANES, LANES)]
                lo = lax.bitcast_convert_type(v << 16, jnp.float32)
                hi = lax.bitcast_convert_type(v & HMASK, jnp.float32)
                sle = pl.ds(c * LANES, LANES)
                slo = pl.ds(H + c * LANES, LANES)
                rows_b[r, sle] = rows_b[r, sle] * lo
                rows_b[r, slo] = rows_b[r, slo] * hi

        def mul(b):
            rows_b, rad_b = rows_s[b], rad_s[b]

            @plsc.parallel_loop(0, G, 1, unroll=8)
            def _(r):
                mul_row(rows_b, rad_b, r)

        def snap(b):
            src, dst = cidx_s[b], scidx_s[b]

            @plsc.parallel_loop(0, G // LANES, 1, unroll=2)
            def _(k):
                sl = pl.ds(k * LANES, LANES)
                dst[sl] = src[sl]

        def issue_scat(b):
            pltpu.async_copy(rows_s[b], macc.at[scidx_s[b]], sem_s[b],
                             add=True)

        def wait_scat(b):
            pltpu.make_async_copy(rows_s[b], macc.at[scidx_s[b]],
                                  sem_s[b]).wait()

        zero = jnp.zeros((LANES,), jnp.float32)

        def zfill(r, carry):
            for c in range(D // LANES):
                rows0[r, pl.ds(c * LANES, LANES)] = zero
            return carry
        lax.fori_loop(0, G, zfill, 0)

        def zcopy(k, carry):
            pltpu.sync_copy(rows0, macc.at[pl.ds(sid * RPT + k * G, G)])
            return carry
        lax.fori_loop(0, ZF, zcopy, 0)
        pltpu.sync_copy(rows0.at[pl.ds(0, ZREM)],
                        macc.at[pl.ds(sid * RPT + ZF * G, ZREM)])

        @pl.when(sid == 0)
        def _():
            pltpu.sync_copy(rows0.at[pl.ds(0, REM)],
                            macc.at[pl.ds(RPT * NS, REM)])

        plsc.subcore_barrier()

        def ops(g, b, has_idx=True, has_fetch=True, has_drain=True):
            bf = (b + 2) % NSLOT
            wait_fetch(b)
            snap(b)
            if has_idx:
                issue_idx(g + NSLOT, b)
            if has_fetch:
                wait_idx(bf)
            if has_drain:
                wait_scat(bf)
            if has_fetch:
                issue_fetch(g + 2, bf)
            mul(b)
            issue_scat(b)

        issue_idx(0, 0)
        issue_idx(1, 1)
        issue_idx(2, 2)
        wait_idx(0)
        issue_fetch(0, 0)
        wait_idx(1)
        issue_fetch(1, 1)
        ops(0, 0, has_drain=False)
        ops(1, 1)
        ops(2, 2)

        def trio(p, carry):
            ops(3 * p, 0)
            ops(3 * p + 1, 1)
            ops(3 * p + 2, 2)
            return carry
        lax.fori_loop(1, (NG - 3) // 3, trio, 0)

        ops(NG - 3, 0, has_idx=False)
        ops(NG - 2, 1, has_idx=False, has_fetch=False)
        ops(NG - 1, 2, has_idx=False, has_fetch=False)

        toff = base0 + NG * G
        pltpu.sync_copy(ec_hbm.at[pl.ds(toff, TAIL)], tcidx)
        pltpu.sync_copy(en_hbm.at[pl.ds(toff, TAIL)], tnidx)
        pltpu.async_copy(node_hbm.at[tnidx], rows0.at[pl.ds(0, TAIL)],
                         sem_g[0]).wait()
        pltpu.sync_copy(rad_hbm.at[pl.ds(toff, TAIL)],
                        rad0.at[pl.ds(0, TAIL)])

        def tmul(r, carry):
            mul_row(rows0, rad0, r)
            return carry
        lax.fori_loop(0, TAIL, tmul, 0)
        pltpu.sync_copy(rows0.at[pl.ds(0, TAIL)], macc.at[tcidx], add=True)

        wait_scat(2)
        plsc.subcore_barrier()

        r0 = sid * RPT
        pltpu.sync_copy(macc.at[pl.ds(r0, RPT)],
                        out_hbm.at[cid, pl.ds(r0, RPT)])

        @pl.when(sid == 0)
        def _():
            pltpu.sync_copy(macc.at[pl.ds(RPT * NS, REM)],
                            out_hbm.at[cid, pl.ds(RPT * NS, REM)])

    return sck(node_attr, edge_center, edge_neigh, proj_radial)


def _combine_body(na_ref, p_ref, wc_ref, wt_ref, wb_ref, g_ref, b_ref, o_ref):
    a = na_ref[...] * wc_ref[...]
    m = p_ref[0] + p_ref[1]
    x = (jnp.dot(a, wt_ref[...], preferred_element_type=jnp.float32)
         + jnp.dot(m, wb_ref[...], preferred_element_type=jnp.float32))
    mu = jnp.mean(x, axis=-1, keepdims=True)
    xc = x - mu
    var = jnp.mean(xc * xc, axis=-1, keepdims=True)
    o_ref[...] = xc * lax.rsqrt(var + 1e-5) * g_ref[...] + b_ref[...]


def _combine(node_attr, partials, wc_diag, W_top, W_bot, gamma, beta):
    N, D = node_attr.shape
    BN = 400
    assert N % BN == 0
    return pl.pallas_call(
        _combine_body,
        grid=(N // BN,),
        in_specs=[
            pl.BlockSpec((BN, D), lambda i: (i, 0)),
            pl.BlockSpec((NC, BN, D), lambda i: (0, i, 0)),
            pl.BlockSpec((1, D), lambda i: (0, 0)),
            pl.BlockSpec((D, D), lambda i: (0, 0)),
            pl.BlockSpec((D, D), lambda i: (0, 0)),
            pl.BlockSpec((1, D), lambda i: (0, 0)),
            pl.BlockSpec((1, D), lambda i: (0, 0)),
        ],
        out_specs=pl.BlockSpec((BN, D), lambda i: (i, 0)),
        out_shape=jax.ShapeDtypeStruct((N, D), jnp.float32),
    )(node_attr, partials, wc_diag.reshape(1, D), W_top, W_bot,
      gamma.reshape(1, D), beta.reshape(1, D))


def kernel(node_attr, edge_index, edge_radial_attrs, W_node, W_center_node,
           W_concat_node, W_edge, ln_gamma, ln_beta):
    D = node_attr.shape[1]
    wnode = jnp.diagonal(W_node)
    wcenter = jnp.diagonal(W_center_node)
    perm = jnp.concatenate([jnp.arange(0, D, 2), jnp.arange(1, D, 2)])
    node_perm = node_attr[:, perm]
    Wp = (W_edge * wnode[None, :])[:, perm]
    rad_packed = _proj_radial(edge_radial_attrs, Wp)
    ei = edge_index.astype(jnp.int32)
    partials = _sc_edge_aggregate(node_perm, ei[0], ei[1], rad_packed)
    W_top = W_concat_node[:D]
    W_bot = W_concat_node[D:][perm, :]
    return _combine(node_attr, partials, wcenter, W_top, W_bot,
                    ln_gamma, ln_beta)


# speedup vs baseline: 1.1083x; 1.0341x over previous
import functools

import jax
import jax.numpy as jnp
from jax import lax
from jax.experimental import pallas as pl
from jax.experimental.pallas import tpu as pltpu
from jax.experimental.pallas import tpu_sc as plsc

NC = 2
NS = 16
LANES = 16


def _radial_body(r_ref, w_ref, o_ref):
    x = jnp.dot(r_ref[...], w_ref[...], preferred_element_type=jnp.float32)
    h = x.shape[1] // 2
    u = lax.bitcast_convert_type(x, jnp.uint32) + jnp.uint32(0x8000)
    lo = u[:, :h] >> 16
    hi = u[:, h:] & jnp.uint32(0xFFFF0000)
    o_ref[...] = lax.bitcast_convert_type(lo | hi, jnp.int32)


def _proj_radial(R, W):
    E, DE = R.shape
    D = W.shape[1]
    BE = 4000
    assert E % BE == 0 and D % 2 == 0
    return pl.pallas_call(
        _radial_body,
        grid=(E // BE,),
        in_specs=[
            pl.BlockSpec((BE, DE), lambda i: (i, 0)),
            pl.BlockSpec((DE, D), lambda i: (0, 0)),
        ],
        out_specs=pl.BlockSpec((BE, D // 2), lambda i: (i, 0)),
        out_shape=jax.ShapeDtypeStruct((E, D // 2), jnp.int32),
    )(R, W)


def _sc_edge_aggregate(node_attr, edge_center, edge_neigh, proj_radial):
    N, D = node_attr.shape
    E = edge_center.shape[0]
    NW = NC * NS
    EPW = E // NW
    G = 64
    NG = EPW // G
    TAIL = EPW - NG * G
    assert (NG >= 6 and (NG - 6) % 3 == 0 and 0 < TAIL <= G
            and TAIL % 8 == 0 and G % LANES == 0)
    RPT = (N // NS) // 8 * 8
    REM = N - RPT * NS
    ZF = RPT // G
    ZREM = RPT - ZF * G
    assert REM % 8 == 0 and ZREM % 8 == 0 and (RPT * NS) % 8 == 0

    mesh = plsc.VectorSubcoreMesh(core_axis_name="c", subcore_axis_name="s",
                                  num_cores=NC, num_subcores=NS)

    NSLOT = 3
    scratch = []
    for _ in range(NSLOT):
        scratch += [
            pltpu.VMEM((G,), jnp.int32),
            pltpu.VMEM((G,), jnp.int32),
            pltpu.VMEM((G,), jnp.int32),
            pltpu.VMEM((G, D), jnp.float32),
            pltpu.VMEM((G, D // 2), jnp.int32),
        ]
    scratch += [
        pltpu.VMEM((TAIL,), jnp.int32),
        pltpu.VMEM((TAIL,), jnp.int32),
        pltpu.VMEM_SHARED((N, D), jnp.float32),
    ]
    scratch += [pltpu.SemaphoreType.DMA] * (4 * NSLOT)

    @functools.partial(
        pl.kernel,
        out_type=jax.ShapeDtypeStruct((NC, N, D), jnp.float32),
        mesh=mesh,
        scratch_types=scratch,
    )
    def sck(node_hbm, ec_hbm, en_hbm, rad_hbm, out_hbm, *sc):
        slot_refs, rest = sc[:5 * NSLOT], sc[5 * NSLOT:]
        nidx_s = [slot_refs[5 * i + 0] for i in range(NSLOT)]
        cidx_s = [slot_refs[5 * i + 1] for i in range(NSLOT)]
        scidx_s = [slot_refs[5 * i + 2] for i in range(NSLOT)]
        rows_s = [slot_refs[5 * i + 3] for i in range(NSLOT)]
        rad_s = [slot_refs[5 * i + 4] for i in range(NSLOT)]
        tnidx, tcidx, macc = rest[0], rest[1], rest[2]
        sems = rest[3:]
        sem_i = list(sems[0:NSLOT])
        sem_g = list(sems[NSLOT:2 * NSLOT])
        sem_r = list(sems[2 * NSLOT:3 * NSLOT])
        sem_s = list(sems[3 * NSLOT:4 * NSLOT])
        rows0, rad0 = rows_s[0], rad_s[0]
        cid = lax.axis_index("c")
        sid = lax.axis_index("s")
        wid = sid * NC + cid
        base0 = wid * EPW

        def issue_idx(g, b):
            off = base0 + g * G
            pltpu.async_copy(ec_hbm.at[pl.ds(off, G)], cidx_s[b], sem_i[b])
            pltpu.async_copy(en_hbm.at[pl.ds(off, G)], nidx_s[b], sem_i[b])

        def wait_idx(b):
            pltpu.make_async_copy(ec_hbm.at[pl.ds(0, G)], cidx_s[b],
                                  sem_i[b]).wait()
            pltpu.make_async_copy(en_hbm.at[pl.ds(0, G)], nidx_s[b],
                                  sem_i[b]).wait()

        def issue_fetch(g, b):
            pltpu.async_copy(node_hbm.at[nidx_s[b]], rows_s[b], sem_g[b])
            pltpu.async_copy(rad_hbm.at[pl.ds(base0 + g * G, G)], rad_s[b],
                             sem_r[b])

        def wait_fetch(b):
            pltpu.make_async_copy(node_hbm.at[nidx_s[b]], rows_s[b],
                                  sem_g[b]).wait()
            pltpu.make_async_copy(rad_hbm.at[pl.ds(0, G)], rad_s[b],
                                  sem_r[b]).wait()

        HMASK = jnp.full((LANES,), -65536, jnp.int32)
        H = D // 2

        def mul_row(rows_b, rad_b, r):
            for c in range(H // LANES):
                v = rad_b[r, pl.ds(c * LANES, LANES)]
                lo = lax.bitcast_convert_type(v << 16, jnp.float32)
                hi = lax.bitcast_convert_type(v & HMASK, jnp.float32)
                sle = pl.ds(c * LANES, LANES)
                slo = pl.ds(H + c * LANES, LANES)
                rows_b[r, sle] = rows_b[r, sle] * lo
                rows_b[r, slo] = rows_b[r, slo] * hi

        def mul(b):
            rows_b, rad_b = rows_s[b], rad_s[b]

            @plsc.parallel_loop(0, G, 1, unroll=8)
            def _(r):
                mul_row(rows_b, rad_b, r)

        def snap(b):
            src, dst = cidx_s[b], scidx_s[b]

            @plsc.parallel_loop(0, G // LANES, 1, unroll=2)
            def _(k):
                sl = pl.ds(k * LANES, LANES)
                dst[sl] = src[sl]

        def issue_scat(b):
            pltpu.async_copy(rows_s[b], macc.at[scidx_s[b]], sem_s[b],
                             add=True)

        def wait_scat(b):
            pltpu.make_async_copy(rows_s[b], macc.at[scidx_s[b]],
                                  sem_s[b]).wait()

        zero = jnp.zeros((LANES,), jnp.float32)

        def zfill(r, carry):
            for c in range(D // LANES):
                rows0[r, pl.ds(c * LANES, LANES)] = zero
            return carry
        lax.fori_loop(0, G, zfill, 0)

        def zcopy(k, carry):
            pltpu.sync_copy(rows0, macc.at[pl.ds(sid * RPT + k * G, G)])
            return carry
        lax.fori_loop(0, ZF, zcopy, 0)
        pltpu.sync_copy(rows0.at[pl.ds(0, ZREM)],
                        macc.at[pl.ds(sid * RPT + ZF * G, ZREM)])

        @pl.when(sid == 0)
        def _():
            pltpu.sync_copy(rows0.at[pl.ds(0, REM)],
                            macc.at[pl.ds(RPT * NS, REM)])

        plsc.subcore_barrier()

        def ops(g, b, has_idx=True, has_fetch=True, has_drain=True):
            bf = (b + 2) % NSLOT
            wait_fetch(b)
            snap(b)
            if has_idx:
                issue_idx(g + NSLOT, b)
            if has_fetch:
                wait_idx(bf)
            if has_drain:
                wait_scat(bf)
            if has_fetch:
                issue_fetch(g + 2, bf)
            mul(b)
            issue_scat(b)

        issue_idx(0, 0)
        issue_idx(1, 1)
        issue_idx(2, 2)
        wait_idx(0)
        issue_fetch(0, 0)
        wait_idx(1)
        issue_fetch(1, 1)
        ops(0, 0, has_drain=False)
        ops(1, 1)
        ops(2, 2)

        def trio(p, carry):
            ops(3 * p, 0)
            ops(3 * p + 1, 1)
            ops(3 * p + 2, 2)
            return carry
        lax.fori_loop(1, (NG - 3) // 3, trio, 0)

        ops(NG - 3, 0, has_idx=False)
        ops(NG - 2, 1, has_idx=False, has_fetch=False)
        ops(NG - 1, 2, has_idx=False, has_fetch=False)

        toff = base0 + NG * G
        pltpu.sync_copy(ec_hbm.at[pl.ds(toff, TAIL)], tcidx)
        pltpu.sync_copy(en_hbm.at[pl.ds(toff, TAIL)], tnidx)
        pltpu.async_copy(node_hbm.at[tnidx], rows0.at[pl.ds(0, TAIL)],
                         sem_g[0]).wait()
        pltpu.sync_copy(rad_hbm.at[pl.ds(toff, TAIL)],
                        rad0.at[pl.ds(0, TAIL)])

        def tmul(r, carry):
            mul_row(rows0, rad0, r)
            return carry
        lax.fori_loop(0, TAIL, tmul, 0)
        pltpu.sync_copy(rows0.at[pl.ds(0, TAIL)], macc.at[tcidx], add=True)

        wait_scat(2)
        plsc.subcore_barrier()

        r0 = sid * RPT
        pltpu.sync_copy(macc.at[pl.ds(r0, RPT)],
                        out_hbm.at[cid, pl.ds(r0, RPT)])

        @pl.when(sid == 0)
        def _():
            pltpu.sync_copy(macc.at[pl.ds(RPT * NS, REM)],
                            out_hbm.at[cid, pl.ds(RPT * NS, REM)])

    return sck(node_attr, edge_center, edge_neigh, proj_radial)


def _combine_body(na_ref, p_ref, wc_ref, wt_ref, wb_ref, g_ref, b_ref, o_ref):
    a = na_ref[...] * wc_ref[...]
    m = p_ref[0] + p_ref[1]
    x = (jnp.dot(a, wt_ref[...], preferred_element_type=jnp.float32)
         + jnp.dot(m, wb_ref[...], preferred_element_type=jnp.float32))
    mu = jnp.mean(x, axis=-1, keepdims=True)
    xc = x - mu
    var = jnp.mean(xc * xc, axis=-1, keepdims=True)
    o_ref[...] = xc * lax.rsqrt(var + 1e-5) * g_ref[...] + b_ref[...]


def _combine(node_attr, partials, wc_diag, W_top, W_bot, gamma, beta):
    N, D = node_attr.shape
    BN = 400
    assert N % BN == 0
    return pl.pallas_call(
        _combine_body,
        grid=(N // BN,),
        in_specs=[
            pl.BlockSpec((BN, D), lambda i: (i, 0)),
            pl.BlockSpec((NC, BN, D), lambda i: (0, i, 0)),
            pl.BlockSpec((1, D), lambda i: (0, 0)),
            pl.BlockSpec((D, D), lambda i: (0, 0)),
            pl.BlockSpec((D, D), lambda i: (0, 0)),
            pl.BlockSpec((1, D), lambda i: (0, 0)),
            pl.BlockSpec((1, D), lambda i: (0, 0)),
        ],
        out_specs=pl.BlockSpec((BN, D), lambda i: (i, 0)),
        out_shape=jax.ShapeDtypeStruct((N, D), jnp.float32),
    )(node_attr, partials, wc_diag.reshape(1, D), W_top, W_bot,
      gamma.reshape(1, D), beta.reshape(1, D))


def kernel(node_attr, edge_index, edge_radial_attrs, W_node, W_center_node,
           W_concat_node, W_edge, ln_gamma, ln_beta):
    D = node_attr.shape[1]
    wnode = jnp.diagonal(W_node)
    wcenter = jnp.diagonal(W_center_node)
    rad_packed = _proj_radial(edge_radial_attrs, W_edge * wnode[None, :])
    ei = edge_index.astype(jnp.int32)
    partials = _sc_edge_aggregate(node_attr, ei[0], ei[1], rad_packed)
    return _combine(node_attr, partials, wcenter, W_concat_node[:D],
                    W_concat_node[D:], ln_gamma, ln_beta)
